# baseline (speedup 1.0000x reference)
import functools

import jax
import jax.numpy as jnp
from jax import lax
from jax.experimental import pallas as pl
from jax.experimental.pallas import tpu as pltpu
from jax.experimental.pallas import tpu_sc as plsc

N = 10000
E = 320000
C = 10

CK = 125
NJ = 40
NTILE = 16
NCORE = 2
ROWS = E // CK
RPT = ROWS // NTILE
NG = RPT // NJ
NPAD = 10112
NPT = NPAD // NTILE
ZR = 32
RB = 2000
GRID = N // RB

_HIGH = None


def _dot(a, b):
    return jnp.dot(a, b, preferred_element_type=jnp.float32, precision=_HIGH)


def _make_agg(nb, with_deg):
    nbh = nb // 2
    mesh = plsc.VectorSubcoreMesh(
        core_axis_name="c", subcore_axis_name="s",
        num_cores=NCORE, num_subcores=NTILE)
    out_type = [jax.ShapeDtypeStruct((nb, NPAD, 128), jnp.float32)]
    if with_deg:
        out_type.append(jax.ShapeDtypeStruct((2, NPAD, 128), jnp.float32))
    scratch = [
        pltpu.VMEM((NJ, CK), jnp.int32),
        pltpu.VMEM((NJ, CK), jnp.int32),
        pltpu.VMEM((CK, 128), jnp.float32),
        pltpu.VMEM((CK, 128), jnp.float32),
        pltpu.VMEM((ZR, 128), jnp.float32),
        pltpu.VMEM_SHARED((NPAD, 128), jnp.float32),
        pltpu.SemaphoreType.DMA,
        pltpu.SemaphoreType.DMA,
    ]

    NQ = NJ // 2

    @functools.partial(pl.kernel, out_type=tuple(out_type), mesh=mesh,
                       scratch_types=tuple(scratch))
    def agg(t, srcb, dstb, m_out, *rest):
        if with_deg:
            deg_out = rest[0]
            rest = rest[1:]
        src_blk, dst_blk, rows0, rows1, zbuf, acc, semg, sems = rest[:8]
        c = lax.axis_index("c")
        s = lax.axis_index("s")

        def wait_g():
            pltpu.make_async_copy(t.at[0].at[src_blk.at[0]], rows0, semg).wait()

        def wait_s():
            pltpu.make_async_copy(rows0, acc.at[dst_blk.at[0]], sems).wait()

        def zb(k, carry):
            for jj in range(8):
                zbuf[k, pl.ds(jj * 16, 16)] = jnp.zeros((16,), jnp.float32)
            return carry
        lax.fori_loop(0, ZR, zb, 0)

        def zero_acc():
            for tt in range(NPT // ZR):
                off = s * NPT + tt * ZR
                pltpu.sync_copy(zbuf, acc.at[pl.ds(off, ZR)])
            if NPT % ZR:
                pltpu.sync_copy(zbuf.at[pl.ds(0, NPT % ZR)],
                                acc.at[pl.ds(s * NPT + (NPT // ZR) * ZR,
                                             NPT % ZR)])

        for pj in range(nbh):
            b = c * nbh + pj
            tb = t.at[b]
            zero_acc()
            plsc.subcore_barrier()

            def gbody(g, carry):
                base = s * RPT + g * NJ
                pltpu.sync_copy(srcb.at[pl.ds(base, NJ)], src_blk)
                pltpu.sync_copy(dstb.at[pl.ds(base, NJ)], dst_blk)
                pltpu.async_copy(tb.at[src_blk.at[0]], rows0, semg)
                pltpu.async_copy(tb.at[src_blk.at[1]], rows1, semg)

                def pbody(q, qc):
                    j0 = 2 * q
                    wait_g()
                    pltpu.async_copy(rows0, acc.at[dst_blk.at[j0]], sems,
                                     add=True)
                    wait_g()
                    pltpu.async_copy(rows1, acc.at[dst_blk.at[j0 + 1]], sems,
                                     add=True)
                    wait_s()

                    @pl.when(q < NQ - 1)
                    def _g2():
                        pltpu.async_copy(tb.at[src_blk.at[j0 + 2]], rows0,
                                         semg)
                    wait_s()

                    @pl.when(q < NQ - 1)
                    def _g3():
                        pltpu.async_copy(tb.at[src_blk.at[j0 + 3]], rows1,
                                         semg)
                    return qc
                lax.fori_loop(0, NQ, pbody, 0)
                return carry
            lax.fori_loop(0, NG, gbody, 0)
            plsc.subcore_barrier()

            off = s * NPT
            pltpu.sync_copy(acc.at[pl.ds(off, NPT)],
                            m_out.at[b].at[pl.ds(off, NPT)])
            if with_deg or pj + 1 < nbh:
                plsc.subcore_barrier()

        if with_deg:
            zero_acc()

            def ob(k, carry):
                for jj in range(8):
                    rows0[k, pl.ds(jj * 16, 16)] = jnp.ones((16,), jnp.float32)
                return carry
            lax.fori_loop(0, CK, ob, 0)
            plsc.subcore_barrier()

            def dbody(g, carry):
                @pl.when(g < 2)
                def _do():
                    base = c * 1280 + s * 80 + g * NJ
                    pltpu.sync_copy(dstb.at[pl.ds(base, NJ)], dst_blk)

                    def jfire(j, jc):
                        pltpu.async_copy(rows0, acc.at[dst_blk.at[j]], sems,
                                         add=True)
                        return jc
                    lax.fori_loop(0, NJ, jfire, 0)

                    def jdrain(j, jc):
                        wait_s()
                        return jc
                    lax.fori_loop(0, NJ, jdrain, 0)
                return carry
            lax.fori_loop(0, 3, dbody, 0)
            plsc.subcore_barrier()

            off = s * NPT
            pltpu.sync_copy(acc.at[pl.ds(off, NPT)],
                            deg_out.at[c].at[pl.ds(off, NPT)])

    return agg


_agg2 = _make_agg(2, True)
_agg4 = _make_agg(4, False)


def _interleave(z, t_ref, base):
    z2 = z * z
    t_ref[base] = jnp.concatenate([z[:, :64], z2[:, :64]], axis=1)
    t_ref[base + 1] = jnp.concatenate([z[:, 64:], z2[:, 64:]], axis=1)


def _prep_body(x_ref, t_ref):
    _interleave(x_ref[...], t_ref, 0)


def _prep(x):
    return pl.pallas_call(
        _prep_body,
        grid=(GRID,),
        in_specs=[pl.BlockSpec((RB, 128), lambda i: (i, 0))],
        out_specs=pl.BlockSpec((2, RB, 128), lambda i: (0, i, 0)),
        out_shape=jax.ShapeDtypeStruct((2, N, 128), jnp.float32),
    )(x)


def _mlp_zs(m_ref, deg_ref, w0_ref, b0_ref, w1_ref, b1_ref, nb):
    deg = deg_ref[0][:, 0:1] + deg_ref[1][:, 0:1]
    dinv = 1.0 / jnp.maximum(deg, 1.0)
    zs = []
    for p in range(2):
        cols = (slice(None, 64), slice(64, None))[p]
        mp = jnp.concatenate([m_ref[bb][:, cols] for bb in range(nb)], axis=1)
        mp = mp * dinv
        z = jnp.maximum(_dot(mp, w0_ref[p]) + b0_ref[p][None, :], 0.0)
        z = jnp.maximum(_dot(z, w1_ref[p]) + b1_ref[p][None, :], 0.0)
        zs.append(z)
    return zs


def _make_mlp(nb):
    def body(m_ref, deg_ref, w0_ref, b0_ref, w1_ref, b1_ref, t_ref):
        zs = _mlp_zs(m_ref, deg_ref, w0_ref, b0_ref, w1_ref, b1_ref, nb)
        for p in range(2):
            _interleave(zs[p], t_ref, 2 * p)

    fin = 64 * nb

    def call(m, deg16, w0, b0, w1, b1):
        return pl.pallas_call(
            body,
            grid=(GRID,),
            in_specs=[
                pl.BlockSpec((nb, RB, 128), lambda i: (0, i, 0)),
                pl.BlockSpec((2, RB, 128), lambda i: (0, i, 0)),
                pl.BlockSpec((2, fin, 128), lambda i: (0, 0, 0)),
                pl.BlockSpec((2, 128), lambda i: (0, 0)),
                pl.BlockSpec((2, 128, 128), lambda i: (0, 0, 0)),
                pl.BlockSpec((2, 128), lambda i: (0, 0)),
            ],
            out_specs=pl.BlockSpec((4, RB, 128), lambda i: (0, i, 0)),
            out_shape=jax.ShapeDtypeStruct((4, N, 128), jnp.float32),
        )(m, deg16, w0, b0, w1, b1)

    return call


_mlp0 = _make_mlp(2)
_mlp12 = _make_mlp(4)


def _mlpl_body(m_ref, deg_ref, w0_ref, b0_ref, w1_ref, b1_ref, wc_ref, bc_ref,
               out_ref, acc_ref):
    i = pl.program_id(0)
    zs = _mlp_zs(m_ref, deg_ref, w0_ref, b0_ref, w1_ref, b1_ref, 4)
    ssum = jnp.sum(jnp.concatenate(zs, axis=1), axis=0, keepdims=True)

    @pl.when(i == 0)
    def _init():
        acc_ref[...] = jnp.zeros_like(acc_ref)

    acc_ref[...] += ssum

    @pl.when(i == GRID - 1)
    def _fin():
        hg = acc_ref[...] * (1.0 / N)
        out_ref[...] = _dot(hg, wc_ref[...]) + bc_ref[...]


def _mlp_last(m, deg16, w0, b0, w1, b1, wc, bc):
    return pl.pallas_call(
        _mlpl_body,
        grid=(GRID,),
        in_specs=[
            pl.BlockSpec((4, RB, 128), lambda i: (0, i, 0)),
            pl.BlockSpec((2, RB, 128), lambda i: (0, i, 0)),
            pl.BlockSpec((2, 256, 128), lambda i: (0, 0, 0)),
            pl.BlockSpec((2, 128), lambda i: (0, 0)),
            pl.BlockSpec((2, 128, 128), lambda i: (0, 0, 0)),
            pl.BlockSpec((2, 128), lambda i: (0, 0)),
            pl.BlockSpec((256, C), lambda i: (0, 0)),
            pl.BlockSpec((1, C), lambda i: (0, 0)),
        ],
        out_specs=pl.BlockSpec((1, C), lambda i: (0, 0)),
        out_shape=jax.ShapeDtypeStruct((1, C), jnp.float32),
        scratch_shapes=[pltpu.VMEM((1, 256), jnp.float32)],
    )(m, deg16, w0, b0, w1, b1, wc, bc)


def kernel(x, edge_index, W_l0, b_l0, W_l12_g0, b_l12_g0, W_l12_g1, b_l12_g1,
           Wc, bc):
    src = edge_index[0].astype(jnp.int32).reshape(ROWS, CK)
    dst = edge_index[1].astype(jnp.int32).reshape(ROWS, CK)
    t0 = _prep(x)
    m0, deg16 = _agg2(t0, src, dst)
    t = _mlp0(m0, deg16, W_l0[:, 0], b_l0[:, 0], W_l0[:, 1], b_l0[:, 1])
    for l in range(2):
        (m,) = _agg4(t, src, dst)
        if l == 0:
            t = _mlp12(m, deg16, W_l12_g0[0], b_l12_g0[0],
                       W_l12_g1[0], b_l12_g1[0])
        else:
            logits = _mlp_last(m, deg16, W_l12_g0[1], b_l12_g0[1],
                               W_l12_g1[1], b_l12_g1[1], Wc,
                               bc.reshape(1, C))
    return logits


# speedup vs baseline: 1.0465x; 1.0465x over previous
import functools

import jax
import jax.numpy as jnp
from jax import lax
from jax.experimental import pallas as pl
from jax.experimental.pallas import tpu as pltpu
from jax.experimental.pallas import tpu_sc as plsc

N = 10000
E = 320000
C = 10

CK = 125
NJ = 40
NTILE = 16
NCORE = 2
ROWS = E // CK
RPT = ROWS // NTILE
NG = RPT // NJ
NPAD = 10112
NPT = NPAD // NTILE
ZR = 32
RB = 2000
GRID = N // RB

_HIGH = None


def _dot(a, b):
    return jnp.dot(a, b, preferred_element_type=jnp.float32, precision=_HIGH)


def _make_agg(nb, with_deg):
    nbh = nb // 2
    mesh = plsc.VectorSubcoreMesh(
        core_axis_name="c", subcore_axis_name="s",
        num_cores=NCORE, num_subcores=NTILE)
    out_type = [jax.ShapeDtypeStruct((nb, NPAD, 128), jnp.float32)]
    if with_deg:
        out_type.append(jax.ShapeDtypeStruct((2, NPAD, 128), jnp.float32))
    scratch = [
        pltpu.VMEM((NJ, CK), jnp.int32),
        pltpu.VMEM((NJ, CK), jnp.int32),
        pltpu.VMEM((CK, 128), jnp.float32),
        pltpu.VMEM((CK, 128), jnp.float32),
        pltpu.VMEM((ZR, 128), jnp.float32),
        pltpu.VMEM_SHARED((NPAD, 128), jnp.float32),
        pltpu.SemaphoreType.DMA,
        pltpu.SemaphoreType.DMA,
    ]

    NQ = NJ // 2

    @functools.partial(pl.kernel, out_type=tuple(out_type), mesh=mesh,
                       scratch_types=tuple(scratch))
    def agg(t, srcb, dstb, m_out, *rest):
        if with_deg:
            deg_out = rest[0]
            rest = rest[1:]
        src_blk, dst_blk, rows0, rows1, zbuf, acc, semg, sems = rest[:8]
        c = lax.axis_index("c")
        s = lax.axis_index("s")

        def wait_g():
            pltpu.make_async_copy(t.at[0].at[src_blk.at[0]], rows0, semg).wait()

        def wait_s():
            pltpu.make_async_copy(rows0, acc.at[dst_blk.at[0]], sems).wait()

        def zb(k, carry):
            for jj in range(8):
                zbuf[k, pl.ds(jj * 16, 16)] = jnp.zeros((16,), jnp.float32)
            return carry
        lax.fori_loop(0, ZR, zb, 0)

        def zero_acc():
            for tt in range(NPT // ZR):
                off = s * NPT + tt * ZR
                pltpu.sync_copy(zbuf, acc.at[pl.ds(off, ZR)])
            if NPT % ZR:
                pltpu.sync_copy(zbuf.at[pl.ds(0, NPT % ZR)],
                                acc.at[pl.ds(s * NPT + (NPT // ZR) * ZR,
                                             NPT % ZR)])

        for pj in range(nbh):
            b = c * nbh + pj
            tb = t.at[b]
            zero_acc()
            plsc.subcore_barrier()

            def gbody(g, carry):
                base = s * RPT + g * NJ
                pltpu.sync_copy(srcb.at[pl.ds(base, NJ)], src_blk)
                pltpu.sync_copy(dstb.at[pl.ds(base, NJ)], dst_blk)
                pltpu.async_copy(tb.at[src_blk.at[0]], rows0, semg)

                def pbody(q, qc):
                    j0 = 2 * q

                    @pl.when(q > 0)
                    def _free1():
                        wait_s()
                    pltpu.async_copy(tb.at[src_blk.at[j0 + 1]], rows1, semg)
                    wait_g()
                    pltpu.async_copy(rows0, acc.at[dst_blk.at[j0]], sems,
                                     add=True)
                    wait_s()

                    @pl.when(q < NQ - 1)
                    def _g2():
                        pltpu.async_copy(tb.at[src_blk.at[j0 + 2]], rows0,
                                         semg)
                    wait_g()
                    pltpu.async_copy(rows1, acc.at[dst_blk.at[j0 + 1]], sems,
                                     add=True)
                    return qc
                lax.fori_loop(0, NQ, pbody, 0)
                wait_s()
                return carry
            lax.fori_loop(0, NG, gbody, 0)
            plsc.subcore_barrier()

            off = s * NPT
            pltpu.sync_copy(acc.at[pl.ds(off, NPT)],
                            m_out.at[b].at[pl.ds(off, NPT)])
            if with_deg or pj + 1 < nbh:
                plsc.subcore_barrier()

        if with_deg:
            zero_acc()

            def ob(k, carry):
                for jj in range(8):
                    rows0[k, pl.ds(jj * 16, 16)] = jnp.ones((16,), jnp.float32)
                return carry
            lax.fori_loop(0, CK, ob, 0)
            plsc.subcore_barrier()

            def dbody(g, carry):
                @pl.when(g < 2)
                def _do():
                    base = c * 1280 + s * 80 + g * NJ
                    pltpu.sync_copy(dstb.at[pl.ds(base, NJ)], dst_blk)

                    def jfire(j, jc):
                        pltpu.async_copy(rows0, acc.at[dst_blk.at[j]], sems,
                                         add=True)
                        return jc
                    lax.fori_loop(0, NJ, jfire, 0)

                    def jdrain(j, jc):
                        wait_s()
                        return jc
                    lax.fori_loop(0, NJ, jdrain, 0)
                return carry
            lax.fori_loop(0, 3, dbody, 0)
            plsc.subcore_barrier()

            off = s * NPT
            pltpu.sync_copy(acc.at[pl.ds(off, NPT)],
                            deg_out.at[c].at[pl.ds(off, NPT)])

    return agg


_agg2 = _make_agg(2, True)
_agg4 = _make_agg(4, False)


def _interleave(z, t_ref, base):
    z2 = z * z
    t_ref[base] = jnp.concatenate([z[:, :64], z2[:, :64]], axis=1)
    t_ref[base + 1] = jnp.concatenate([z[:, 64:], z2[:, 64:]], axis=1)


def _prep_body(x_ref, t_ref):
    _interleave(x_ref[...], t_ref, 0)


def _prep(x):
    return pl.pallas_call(
        _prep_body,
        grid=(GRID,),
        in_specs=[pl.BlockSpec((RB, 128), lambda i: (i, 0))],
        out_specs=pl.BlockSpec((2, RB, 128), lambda i: (0, i, 0)),
        out_shape=jax.ShapeDtypeStruct((2, N, 128), jnp.float32),
    )(x)


def _mlp_zs(m_ref, deg_ref, w0_ref, b0_ref, w1_ref, b1_ref, nb):
    deg = deg_ref[0][:, 0:1] + deg_ref[1][:, 0:1]
    dinv = 1.0 / jnp.maximum(deg, 1.0)
    zs = []
    for p in range(2):
        cols = (slice(None, 64), slice(64, None))[p]
        mp = jnp.concatenate([m_ref[bb][:, cols] for bb in range(nb)], axis=1)
        mp = mp * dinv
        z = jnp.maximum(_dot(mp, w0_ref[p]) + b0_ref[p][None, :], 0.0)
        z = jnp.maximum(_dot(z, w1_ref[p]) + b1_ref[p][None, :], 0.0)
        zs.append(z)
    return zs


def _make_mlp(nb):
    def body(m_ref, deg_ref, w0_ref, b0_ref, w1_ref, b1_ref, t_ref):
        zs = _mlp_zs(m_ref, deg_ref, w0_ref, b0_ref, w1_ref, b1_ref, nb)
        for p in range(2):
            _interleave(zs[p], t_ref, 2 * p)

    fin = 64 * nb

    def call(m, deg16, w0, b0, w1, b1):
        return pl.pallas_call(
            body,
            grid=(GRID,),
            in_specs=[
                pl.BlockSpec((nb, RB, 128), lambda i: (0, i, 0)),
                pl.BlockSpec((2, RB, 128), lambda i: (0, i, 0)),
                pl.BlockSpec((2, fin, 128), lambda i: (0, 0, 0)),
                pl.BlockSpec((2, 128), lambda i: (0, 0)),
                pl.BlockSpec((2, 128, 128), lambda i: (0, 0, 0)),
                pl.BlockSpec((2, 128), lambda i: (0, 0)),
            ],
            out_specs=pl.BlockSpec((4, RB, 128), lambda i: (0, i, 0)),
            out_shape=jax.ShapeDtypeStruct((4, N, 128), jnp.float32),
        )(m, deg16, w0, b0, w1, b1)

    return call


_mlp0 = _make_mlp(2)
_mlp12 = _make_mlp(4)


def _mlpl_body(m_ref, deg_ref, w0_ref, b0_ref, w1_ref, b1_ref, wc_ref, bc_ref,
               out_ref, acc_ref):
    i = pl.program_id(0)
    zs = _mlp_zs(m_ref, deg_ref, w0_ref, b0_ref, w1_ref, b1_ref, 4)
    ssum = jnp.sum(jnp.concatenate(zs, axis=1), axis=0, keepdims=True)

    @pl.when(i == 0)
    def _init():
        acc_ref[...] = jnp.zeros_like(acc_ref)

    acc_ref[...] += ssum

    @pl.when(i == GRID - 1)
    def _fin():
        hg = acc_ref[...] * (1.0 / N)
        out_ref[...] = _dot(hg, wc_ref[...]) + bc_ref[...]


def _mlp_last(m, deg16, w0, b0, w1, b1, wc, bc):
    return pl.pallas_call(
        _mlpl_body,
        grid=(GRID,),
        in_specs=[
            pl.BlockSpec((4, RB, 128), lambda i: (0, i, 0)),
            pl.BlockSpec((2, RB, 128), lambda i: (0, i, 0)),
            pl.BlockSpec((2, 256, 128), lambda i: (0, 0, 0)),
            pl.BlockSpec((2, 128), lambda i: (0, 0)),
            pl.BlockSpec((2, 128, 128), lambda i: (0, 0, 0)),
            pl.BlockSpec((2, 128), lambda i: (0, 0)),
            pl.BlockSpec((256, C), lambda i: (0, 0)),
            pl.BlockSpec((1, C), lambda i: (0, 0)),
        ],
        out_specs=pl.BlockSpec((1, C), lambda i: (0, 0)),
        out_shape=jax.ShapeDtypeStruct((1, C), jnp.float32),
        scratch_shapes=[pltpu.VMEM((1, 256), jnp.float32)],
    )(m, deg16, w0, b0, w1, b1, wc, bc)


def kernel(x, edge_index, W_l0, b_l0, W_l12_g0, b_l12_g0, W_l12_g1, b_l12_g1,
           Wc, bc):
    src = edge_index[0].astype(jnp.int32).reshape(ROWS, CK)
    dst = edge_index[1].astype(jnp.int32).reshape(ROWS, CK)
    t0 = _prep(x)
    m0, deg16 = _agg2(t0, src, dst)
    t = _mlp0(m0, deg16, W_l0[:, 0], b_l0[:, 0], W_l0[:, 1], b_l0[:, 1])
    for l in range(2):
        (m,) = _agg4(t, src, dst)
        if l == 0:
            t = _mlp12(m, deg16, W_l12_g0[0], b_l12_g0[0],
                       W_l12_g1[0], b_l12_g1[0])
        else:
            logits = _mlp_last(m, deg16, W_l12_g0[1], b_l12_g0[1],
                               W_l12_g1[1], b_l12_g1[1], Wc,
                               bc.reshape(1, C))
    return logits
